# baseline (speedup 1.0000x reference)
import functools

import jax
import jax.numpy as jnp
from jax import lax
from jax.experimental import pallas as pl
from jax.experimental.pallas import tpu as pltpu
from jax.experimental.pallas import tpu_sc as plsc

_B = 4096
_L = 50
_HID = 128

_NC = 2
_NS = 16
_NW = _NC * _NS
_CB = 2
_CHUNK = _CB * _L
_NB = 4
_K = 4

_mesh = plsc.VectorSubcoreMesh(core_axis_name="c", subcore_axis_name="s")


@functools.lru_cache(maxsize=None)
def _make_gather(nbatch):
    bpw = nbatch // _NW
    nchunk = bpw // _CB
    ngrp = nchunk // _NB

    @functools.partial(
        pl.kernel,
        mesh=_mesh,
        out_type=jax.ShapeDtypeStruct((nbatch, _L, _HID), jnp.float32),
        scratch_types=[
            pltpu.VMEM((nchunk, _CHUNK), jnp.int32),
            pltpu.VMEM((_NB, _CHUNK, _HID), jnp.float32),
            pltpu.SemaphoreType.DMA((_NB,)),
            pltpu.SemaphoreType.DMA((_NB,)),
        ],
    )
    def gather_kernel(src_hbm, table_hbm, out_hbm, idx_v, rows_v, gsem, wsem):
        wid = lax.axis_index("s") * _NC + lax.axis_index("c")
        base_b = wid * bpw
        pltpu.sync_copy(src_hbm.at[wid], idx_v)

        def fire_writebacks(c, b):
            bb = base_b + c * _CB
            for j in range(_CB):
                pltpu.async_copy(
                    rows_v.at[b, pl.ds(j * _L, _L)],
                    out_hbm.at[bb + j],
                    wsem.at[b],
                )

        def wait_writebacks(c, b):
            bb = base_b + c * _CB
            for j in range(_CB):
                pltpu.make_async_copy(
                    rows_v.at[b, pl.ds(j * _L, _L)],
                    out_hbm.at[bb + j],
                    wsem.at[b],
                ).wait()

        def wait_gather(c, b):
            pltpu.make_async_copy(
                table_hbm.at[idx_v.at[c]], rows_v.at[b], gsem.at[b]
            ).wait()

        for b in range(_NB):
            pltpu.async_copy(
                table_hbm.at[idx_v.at[b]], rows_v.at[b], gsem.at[b]
            )

        def group(o, carry):
            for b in range(_NB):
                wait_gather(o * _NB + b, b)
                fire_writebacks(o * _NB + b, b)
            for b in range(_NB):
                wait_writebacks(o * _NB + b, b)
                pltpu.async_copy(
                    table_hbm.at[idx_v.at[(o + 1) * _NB + b]],
                    rows_v.at[b],
                    gsem.at[b],
                )
            return carry

        lax.fori_loop(0, ngrp - 1, group, 0)

        for b in range(_NB):
            wait_gather((ngrp - 1) * _NB + b, b)
            fire_writebacks((ngrp - 1) * _NB + b, b)
        for b in range(_NB):
            wait_writebacks((ngrp - 1) * _NB + b, b)

    return gather_kernel


def kernel(source, table):
    nbatch = _B // _K
    nchunk = nbatch * _L // (_NW * _CHUNK)
    src = source.reshape(_K, _NW, nchunk, _CHUNK).astype(jnp.int32)
    gk = _make_gather(nbatch)
    out = jnp.empty((_B, _L, _HID), jnp.float32)
    for k in range(_K):
        out = lax.dynamic_update_slice(out, gk(src[k], table), (k * nbatch, 0, 0))
    return out


# speedup vs baseline: 3.0468x; 3.0468x over previous
import functools

import jax
import jax.numpy as jnp
from jax import lax
from jax.experimental import pallas as pl
from jax.experimental.pallas import tpu as pltpu
from jax.experimental.pallas import tpu_sc as plsc

_B = 4096
_L = 50
_HID = 128

_NC = 2
_NS = 16
_NW = _NC * _NS
_BPW = _B // _NW
_NB = 5
_NGRP = _L // _NB

_mesh = plsc.VectorSubcoreMesh(core_axis_name="c", subcore_axis_name="s")


@functools.partial(
    pl.kernel,
    mesh=_mesh,
    out_type=jax.ShapeDtypeStruct((_L, _B, _HID), jnp.float32),
    scratch_types=[
        pltpu.VMEM((_L, _BPW), jnp.int32),
        pltpu.VMEM((_NB, _BPW, _HID), jnp.float32),
        pltpu.SemaphoreType.DMA((_NB,)),
        pltpu.SemaphoreType.DMA((_NB,)),
    ],
)
def _gather_kernel(src_hbm, table_hbm, out_hbm, idx_v, rows_v, gsem, wsem):
    wid = lax.axis_index("s") * _NC + lax.axis_index("c")
    base = wid * _BPW
    pltpu.sync_copy(src_hbm.at[wid], idx_v)

    def wait_gather(c, b):
        pltpu.make_async_copy(
            table_hbm.at[idx_v.at[c]], rows_v.at[b], gsem.at[b]
        ).wait()

    def writeback(c, b):
        return (rows_v.at[b], out_hbm.at[c, pl.ds(base, _BPW)], wsem.at[b])

    for b in range(_NB):
        pltpu.async_copy(table_hbm.at[idx_v.at[b]], rows_v.at[b], gsem.at[b])

    def group(o, carry):
        for b in range(_NB):
            wait_gather(o * _NB + b, b)
            pltpu.async_copy(*writeback(o * _NB + b, b))
        for b in range(_NB):
            pltpu.make_async_copy(*writeback(o * _NB + b, b)).wait()
            pltpu.async_copy(
                table_hbm.at[idx_v.at[(o + 1) * _NB + b]],
                rows_v.at[b],
                gsem.at[b],
            )
        return carry

    lax.fori_loop(0, _NGRP - 1, group, 0)

    for b in range(_NB):
        wait_gather((_NGRP - 1) * _NB + b, b)
        pltpu.async_copy(*writeback((_NGRP - 1) * _NB + b, b))
    for b in range(_NB):
        pltpu.make_async_copy(*writeback((_NGRP - 1) * _NB + b, b)).wait()


def kernel(source, table):
    src = (
        source.astype(jnp.int32)
        .T.reshape(_L, _NW, _BPW)
        .transpose(1, 0, 2)
    )
    out = _gather_kernel(src, table)
    return out.transpose(1, 0, 2)


# speedup vs baseline: 3.1291x; 1.0270x over previous
import functools

import jax
import jax.numpy as jnp
from jax import lax
from jax.experimental import pallas as pl
from jax.experimental.pallas import tpu as pltpu
from jax.experimental.pallas import tpu_sc as plsc

_B = 4096
_L = 50
_HID = 128

_NC = 2
_NS = 16
_NW = _NC * _NS
_BPW = _B // _NW
_HC = 2
_CHUNK = _BPW // _HC
_NCH = _L * _HC
_NB = 10
_NGRP = _NCH // _NB

_mesh = plsc.VectorSubcoreMesh(core_axis_name="c", subcore_axis_name="s")


@functools.partial(
    pl.kernel,
    mesh=_mesh,
    out_type=jax.ShapeDtypeStruct((_L, _B, _HID), jnp.float32),
    scratch_types=[
        pltpu.VMEM((_L, _BPW), jnp.int32),
        pltpu.VMEM((_NB, _CHUNK, _HID), jnp.float32),
        pltpu.SemaphoreType.DMA((_NB,)),
        pltpu.SemaphoreType.DMA((_NB,)),
    ],
)
def _gather_kernel(src_hbm, table_hbm, out_hbm, idx_v, rows_v, gsem, wsem):
    wid = lax.axis_index("s") * _NC + lax.axis_index("c")
    base = wid * _BPW
    pltpu.sync_copy(src_hbm.at[wid], idx_v)

    def idx_slice(c):
        return idx_v.at[c // _HC, pl.ds((c % _HC) * _CHUNK, _CHUNK)]

    def gather(c, b):
        return (table_hbm.at[idx_slice(c)], rows_v.at[b], gsem.at[b])

    def writeback(c, b):
        return (
            rows_v.at[b],
            out_hbm.at[c // _HC, pl.ds(base + (c % _HC) * _CHUNK, _CHUNK)],
            wsem.at[b],
        )

    for b in range(_NB):
        pltpu.async_copy(*gather(b, b))

    def group(o, carry):
        for b in range(_NB):
            pltpu.make_async_copy(*gather(o * _NB + b, b)).wait()
            pltpu.async_copy(*writeback(o * _NB + b, b))
        for b in range(_NB):
            pltpu.make_async_copy(*writeback(o * _NB + b, b)).wait()
            pltpu.async_copy(*gather((o + 1) * _NB + b, b))
        return carry

    lax.fori_loop(0, _NGRP - 1, group, 0)

    for b in range(_NB):
        pltpu.make_async_copy(*gather((_NGRP - 1) * _NB + b, b)).wait()
        pltpu.async_copy(*writeback((_NGRP - 1) * _NB + b, b))
    for b in range(_NB):
        pltpu.make_async_copy(*writeback((_NGRP - 1) * _NB + b, b)).wait()


def kernel(source, table):
    src = (
        source.astype(jnp.int32)
        .T.reshape(_L, _NW, _BPW)
        .transpose(1, 0, 2)
    )
    out = _gather_kernel(src, table)
    return out.transpose(1, 0, 2)
